# baseline (speedup 1.0000x reference)
import functools

import jax
import jax.numpy as jnp
from jax import lax
from jax.experimental import pallas as pl
from jax.experimental.pallas import tpu as pltpu
from jax.experimental.pallas import tpu_sc as plsc

N = 10000
E = 320000
NC, NS = 2, 16
NW = NC * NS
E_W = E // NW
K = 40
CHUNKS = E_W // K
N_PAD = 10240
R_T = N_PAD // NS


def _sc_scatter(D, Kc, nbuf):
    ch = E_W // Kc
    mesh = plsc.VectorSubcoreMesh(core_axis_name="c", subcore_axis_name="s")

    @functools.partial(
        pl.kernel,
        out_type=jax.ShapeDtypeStruct((NC, N_PAD, D), jnp.float32),
        mesh=mesh,
        compiler_params=pltpu.CompilerParams(use_tc_tiling_on_sc=False),
        scratch_types=[
            pltpu.VMEM((ch, Kc), jnp.int32),
            pltpu.VMEM((ch, Kc), jnp.int32),
            [pltpu.VMEM((Kc, D), jnp.float32) for _ in range(nbuf)],
            pltpu.VMEM_SHARED((N_PAD, D), jnp.float32),
            [pltpu.SemaphoreType.DMA for _ in range(nbuf)],
        ],
    )
    def k(g_hbm, ei_hbm, zeros_hbm, out_hbm, src_v, dst_v, rows_v,
          acc_s, sems):
        c = lax.axis_index("c")
        s = lax.axis_index("s")
        w = s * NC + c
        row0 = pl.multiple_of(s * R_T, 8)
        pltpu.sync_copy(zeros_hbm, acc_s.at[pl.ds(row0, R_T)])
        pltpu.sync_copy(ei_hbm.at[0, w], src_v)
        pltpu.sync_copy(ei_hbm.at[1, w], dst_v)
        plsc.subcore_barrier()

        for b in range(nbuf - 1):
            pltpu.async_copy(g_hbm.at[src_v.at[b]], rows_v[b], sems[b])

        def body(jo, carry):
            for b in range(nbuf):
                j = jo * nbuf + b
                pltpu.make_async_copy(g_hbm.at[src_v.at[j]], rows_v[b],
                                      sems[b]).wait()
                pltpu.sync_copy(rows_v[b], acc_s.at[dst_v.at[j]], add=True)
                jn = j + nbuf - 1
                bn = (b + nbuf - 1) % nbuf

                @pl.when(jn < ch)
                def _():
                    pltpu.async_copy(g_hbm.at[src_v.at[jn]], rows_v[bn],
                                     sems[bn])
            return carry

        lax.fori_loop(0, ch // nbuf, body, 0)
        plsc.subcore_barrier()
        pltpu.sync_copy(acc_s.at[pl.ds(row0, R_T)],
                        out_hbm.at[c, pl.ds(row0, R_T)])

    return k


_DW = 16


def _sc_degree():
    mesh = plsc.VectorSubcoreMesh(core_axis_name="c", subcore_axis_name="s")

    @functools.partial(
        pl.kernel,
        out_type=jax.ShapeDtypeStruct((NC, N_PAD, _DW), jnp.float32),
        mesh=mesh,
        compiler_params=pltpu.CompilerParams(use_tc_tiling_on_sc=False),
        scratch_types=[
            pltpu.VMEM((CHUNKS, K), jnp.int32),
            pltpu.VMEM((K, _DW), jnp.float32),
            pltpu.VMEM_SHARED((N_PAD, _DW), jnp.float32),
        ],
    )
    def k(ei_hbm, ones_hbm, zeros_hbm, out_hbm, dst_v, ones_v, deg_s):
        c = lax.axis_index("c")
        s = lax.axis_index("s")
        w = s * NC + c
        row0 = pl.multiple_of(s * R_T, 8)
        pltpu.sync_copy(zeros_hbm, deg_s.at[pl.ds(row0, R_T)])
        pltpu.sync_copy(ones_hbm, ones_v)
        pltpu.sync_copy(ei_hbm.at[1, w], dst_v)
        plsc.subcore_barrier()

        def body(j, carry):
            pltpu.sync_copy(ones_v, deg_s.at[dst_v.at[j]], add=True)
            return carry

        lax.fori_loop(0, CHUNKS, body, 0)
        plsc.subcore_barrier()
        pltpu.sync_copy(deg_s.at[pl.ds(row0, R_T)],
                        out_hbm.at[c, pl.ds(row0, R_T)])

    return k


_BR = 1000


def _tc0(x, W1):
    def body(x_ref, w_ref, h_ref):
        h_ref[...] = jnp.dot(x_ref[...], w_ref[...],
                             preferred_element_type=jnp.float32)

    return pl.pallas_call(
        body,
        grid=(N // _BR,),
        in_specs=[
            pl.BlockSpec((_BR, 128), lambda i: (i, 0)),
            pl.BlockSpec((128, 128), lambda i: (0, 0)),
        ],
        out_specs=pl.BlockSpec((_BR, 128), lambda i: (i, 0)),
        out_shape=jax.ShapeDtypeStruct((N, 128), jnp.float32),
    )(x, W1)


def _tc1(h1, deg):
    def body(h_ref, d0_ref, d1_ref, g_ref, dinv_ref):
        deg_tot = d0_ref[0][:, 0:1] + d1_ref[0][:, 0:1] + 1.0
        dinv = lax.rsqrt(deg_tot)
        g_ref[...] = h_ref[...] * dinv
        dinv_ref[...] = dinv

    return pl.pallas_call(
        body,
        grid=(N // _BR,),
        in_specs=[
            pl.BlockSpec((_BR, 128), lambda i: (i, 0)),
            pl.BlockSpec((1, _BR, _DW), lambda i: (0, i, 0)),
            pl.BlockSpec((1, _BR, _DW), lambda i: (1, i, 0)),
        ],
        out_specs=[
            pl.BlockSpec((_BR, 128), lambda i: (i, 0)),
            pl.BlockSpec((_BR, 1), lambda i: (i, 0)),
        ],
        out_shape=[
            jax.ShapeDtypeStruct((N, 128), jnp.float32),
            jax.ShapeDtypeStruct((N, 1), jnp.float32),
        ],
    )(h1, deg, deg)


def _tc2(acc1, g1, dinv, b1, W2):
    def body(a0_ref, a1_ref, g_ref, dinv_ref, b_ref, w_ref, g2_ref):
        z = dinv_ref[...] * (a0_ref[0] + a1_ref[0] + g_ref[...]) + b_ref[...]
        z = jnp.maximum(z, 0.0)
        g2_ref[...] = (
            jnp.dot(z, w_ref[...], preferred_element_type=jnp.float32)
            * dinv_ref[...])

    return pl.pallas_call(
        body,
        grid=(N // _BR,),
        in_specs=[
            pl.BlockSpec((1, _BR, 128), lambda i: (0, i, 0)),
            pl.BlockSpec((1, _BR, 128), lambda i: (1, i, 0)),
            pl.BlockSpec((_BR, 128), lambda i: (i, 0)),
            pl.BlockSpec((_BR, 1), lambda i: (i, 0)),
            pl.BlockSpec((1, 128), lambda i: (0, 0)),
            pl.BlockSpec((128, 64), lambda i: (0, 0)),
        ],
        out_specs=pl.BlockSpec((_BR, 64), lambda i: (i, 0)),
        out_shape=jax.ShapeDtypeStruct((N, 64), jnp.float32),
    )(acc1, acc1, g1, dinv, b1, W2)


def _tc3(acc2, g2, dinv, b2, Wfc, bfc):
    def body(a0_ref, a1_ref, g_ref, dinv_ref, b_ref, w_ref, bfc_ref, o_ref):
        z = dinv_ref[...] * (a0_ref[0] + a1_ref[0] + g_ref[...]) + b_ref[...]
        z = jnp.maximum(z, 0.0)
        o_ref[...] = (
            jnp.dot(z, w_ref[...], preferred_element_type=jnp.float32)
            + bfc_ref[...])

    return pl.pallas_call(
        body,
        grid=(N // _BR,),
        in_specs=[
            pl.BlockSpec((1, _BR, 64), lambda i: (0, i, 0)),
            pl.BlockSpec((1, _BR, 64), lambda i: (1, i, 0)),
            pl.BlockSpec((_BR, 64), lambda i: (i, 0)),
            pl.BlockSpec((_BR, 1), lambda i: (i, 0)),
            pl.BlockSpec((1, 64), lambda i: (0, 0)),
            pl.BlockSpec((64, 1), lambda i: (0, 0)),
            pl.BlockSpec((1, 1), lambda i: (0, 0)),
        ],
        out_specs=pl.BlockSpec((_BR, 1), lambda i: (i, 0)),
        out_shape=jax.ShapeDtypeStruct((N, 1), jnp.float32),
    )(acc2, acc2, g2, dinv, b2, Wfc, bfc)


NB1 = 5
NB2 = 5


def kernel(x, edge_index, W1, b1, W2, b2, Wfc, bfc):
    ei = edge_index.reshape(2, NW, CHUNKS, K)
    zeros1 = jnp.zeros((R_T, _DW), jnp.float32)
    ones_k = jnp.ones((K, _DW), jnp.float32)
    zeros128 = jnp.zeros((R_T, 128), jnp.float32)
    zeros64 = jnp.zeros((R_T, 64), jnp.float32)

    deg = _sc_degree()(ei, ones_k, zeros1)
    h1 = _tc0(x, W1)
    g1, dinv = _tc1(h1, deg)
    acc1 = _sc_scatter(128, K, NB1)(g1, ei, zeros128)
    g2 = _tc2(acc1, g1, dinv, b1.reshape(1, 128), W2)
    acc2 = _sc_scatter(64, K, NB2)(g2, ei, zeros64)
    out = _tc3(acc2, g2, dinv, b2.reshape(1, 64), Wfc,
               bfc.reshape(1, 1))
    return out.reshape(-1)


# speedup vs baseline: 1.0621x; 1.0621x over previous
import functools

import jax
import jax.numpy as jnp
from jax import lax
from jax.experimental import pallas as pl
from jax.experimental.pallas import tpu as pltpu
from jax.experimental.pallas import tpu_sc as plsc

N = 10000
E = 320000
NC, NS = 2, 16
NW = NC * NS
E_W = E // NW
K = 40
CHUNKS = E_W // K
N_PAD = 10240
R_T = N_PAD // NS


def _sc_scatter(D, Kc, nbuf):
    ch = E_W // Kc
    mesh = plsc.VectorSubcoreMesh(core_axis_name="c", subcore_axis_name="s")

    @functools.partial(
        pl.kernel,
        out_type=jax.ShapeDtypeStruct((NC, N_PAD, D), jnp.float32),
        mesh=mesh,
        compiler_params=pltpu.CompilerParams(use_tc_tiling_on_sc=False),
        scratch_types=[
            pltpu.VMEM((ch, Kc), jnp.int32),
            pltpu.VMEM((ch, Kc), jnp.int32),
            [pltpu.VMEM((Kc, D), jnp.float32) for _ in range(nbuf)],
            pltpu.VMEM_SHARED((N_PAD, D), jnp.float32),
            [pltpu.SemaphoreType.DMA for _ in range(nbuf)],
        ],
    )
    def k(g_hbm, ei_hbm, zeros_hbm, out_hbm, src_v, dst_v, rows_v,
          acc_s, sems):
        c = lax.axis_index("c")
        s = lax.axis_index("s")
        w = s * NC + c
        row0 = pl.multiple_of(s * R_T, 8)
        pltpu.sync_copy(zeros_hbm, acc_s.at[pl.ds(row0, R_T)])
        pltpu.sync_copy(ei_hbm.at[0, w], src_v)
        pltpu.sync_copy(ei_hbm.at[1, w], dst_v)
        plsc.subcore_barrier()

        for b in range(nbuf - 1):
            pltpu.async_copy(g_hbm.at[src_v.at[b]], rows_v[b], sems[b])

        def body(jo, carry):
            for b in range(nbuf):
                j = jo * nbuf + b
                pltpu.make_async_copy(g_hbm.at[src_v.at[j]], rows_v[b],
                                      sems[b]).wait()
                pltpu.sync_copy(rows_v[b], acc_s.at[dst_v.at[j]], add=True)
                jn = j + nbuf - 1
                bn = (b + nbuf - 1) % nbuf

                @pl.when(jn < ch)
                def _():
                    pltpu.async_copy(g_hbm.at[src_v.at[jn]], rows_v[bn],
                                     sems[bn])
            return carry

        lax.fori_loop(0, ch // nbuf, body, 0)
        plsc.subcore_barrier()
        pltpu.sync_copy(acc_s.at[pl.ds(row0, R_T)],
                        out_hbm.at[c, pl.ds(row0, R_T)])

    return k


_DW = 16


def _sc_degree():
    mesh = plsc.VectorSubcoreMesh(core_axis_name="c", subcore_axis_name="s")

    @functools.partial(
        pl.kernel,
        out_type=jax.ShapeDtypeStruct((NC, N_PAD, _DW), jnp.float32),
        mesh=mesh,
        compiler_params=pltpu.CompilerParams(use_tc_tiling_on_sc=False),
        scratch_types=[
            pltpu.VMEM((CHUNKS, K), jnp.int32),
            pltpu.VMEM((K, _DW), jnp.float32),
            pltpu.VMEM_SHARED((N_PAD, _DW), jnp.float32),
        ],
    )
    def k(ei_hbm, ones_hbm, zeros_hbm, out_hbm, dst_v, ones_v, deg_s):
        c = lax.axis_index("c")
        s = lax.axis_index("s")
        w = s * NC + c
        row0 = pl.multiple_of(s * R_T, 8)
        pltpu.sync_copy(zeros_hbm, deg_s.at[pl.ds(row0, R_T)])
        pltpu.sync_copy(ones_hbm, ones_v)
        pltpu.sync_copy(ei_hbm.at[1, w], dst_v)
        plsc.subcore_barrier()

        def body(j, carry):
            pltpu.sync_copy(ones_v, deg_s.at[dst_v.at[j]], add=True)
            return carry

        lax.fori_loop(0, CHUNKS, body, 0)
        plsc.subcore_barrier()
        pltpu.sync_copy(deg_s.at[pl.ds(row0, R_T)],
                        out_hbm.at[c, pl.ds(row0, R_T)])

    return k


_BR = 1000


def _tc0(x, W1):
    def body(x_ref, w_ref, h_ref):
        h_ref[...] = jnp.dot(x_ref[...], w_ref[...],
                             preferred_element_type=jnp.float32)

    return pl.pallas_call(
        body,
        grid=(N // _BR,),
        in_specs=[
            pl.BlockSpec((_BR, 128), lambda i: (i, 0)),
            pl.BlockSpec((128, 128), lambda i: (0, 0)),
        ],
        out_specs=pl.BlockSpec((_BR, 128), lambda i: (i, 0)),
        out_shape=jax.ShapeDtypeStruct((N, 128), jnp.float32),
    )(x, W1)


def _tc1(h1, deg):
    def body(h_ref, d0_ref, d1_ref, g_ref, dinv_ref):
        deg_tot = d0_ref[0][:, 0:1] + d1_ref[0][:, 0:1] + 1.0
        dinv = lax.rsqrt(deg_tot)
        g_ref[...] = h_ref[...] * dinv
        dinv_ref[...] = dinv

    return pl.pallas_call(
        body,
        grid=(N // _BR,),
        in_specs=[
            pl.BlockSpec((_BR, 128), lambda i: (i, 0)),
            pl.BlockSpec((1, _BR, _DW), lambda i: (0, i, 0)),
            pl.BlockSpec((1, _BR, _DW), lambda i: (1, i, 0)),
        ],
        out_specs=[
            pl.BlockSpec((_BR, 128), lambda i: (i, 0)),
            pl.BlockSpec((_BR, 1), lambda i: (i, 0)),
        ],
        out_shape=[
            jax.ShapeDtypeStruct((N, 128), jnp.float32),
            jax.ShapeDtypeStruct((N, 1), jnp.float32),
        ],
    )(h1, deg, deg)


def _tc2(acc1, g1, dinv, b1, W2):
    def body(a0_ref, a1_ref, g_ref, dinv_ref, b_ref, w_ref, g2_ref):
        z = dinv_ref[...] * (a0_ref[0] + a1_ref[0] + g_ref[...]) + b_ref[...]
        z = jnp.maximum(z, 0.0)
        g2_ref[...] = (
            jnp.dot(z, w_ref[...], preferred_element_type=jnp.float32)
            * dinv_ref[...])

    return pl.pallas_call(
        body,
        grid=(N // _BR,),
        in_specs=[
            pl.BlockSpec((1, _BR, 128), lambda i: (0, i, 0)),
            pl.BlockSpec((1, _BR, 128), lambda i: (1, i, 0)),
            pl.BlockSpec((_BR, 128), lambda i: (i, 0)),
            pl.BlockSpec((_BR, 1), lambda i: (i, 0)),
            pl.BlockSpec((1, 128), lambda i: (0, 0)),
            pl.BlockSpec((128, 64), lambda i: (0, 0)),
        ],
        out_specs=pl.BlockSpec((_BR, 64), lambda i: (i, 0)),
        out_shape=jax.ShapeDtypeStruct((N, 64), jnp.float32),
    )(acc1, acc1, g1, dinv, b1, W2)


def _tc3(acc2, g2, dinv, b2, Wfc, bfc):
    def body(a0_ref, a1_ref, g_ref, dinv_ref, b_ref, w_ref, bfc_ref, o_ref):
        z = dinv_ref[...] * (a0_ref[0] + a1_ref[0] + g_ref[...]) + b_ref[...]
        z = jnp.maximum(z, 0.0)
        o_ref[...] = (
            jnp.dot(z, w_ref[...], preferred_element_type=jnp.float32)
            + bfc_ref[...])

    return pl.pallas_call(
        body,
        grid=(N // _BR,),
        in_specs=[
            pl.BlockSpec((1, _BR, 64), lambda i: (0, i, 0)),
            pl.BlockSpec((1, _BR, 64), lambda i: (1, i, 0)),
            pl.BlockSpec((_BR, 64), lambda i: (i, 0)),
            pl.BlockSpec((_BR, 1), lambda i: (i, 0)),
            pl.BlockSpec((1, 64), lambda i: (0, 0)),
            pl.BlockSpec((64, 1), lambda i: (0, 0)),
            pl.BlockSpec((1, 1), lambda i: (0, 0)),
        ],
        out_specs=pl.BlockSpec((_BR, 1), lambda i: (i, 0)),
        out_shape=jax.ShapeDtypeStruct((N, 1), jnp.float32),
    )(acc2, acc2, g2, dinv, b2, Wfc, bfc)


NB1 = 5
K2, NB2 = 80, 5


def kernel(x, edge_index, W1, b1, W2, b2, Wfc, bfc):
    ei = edge_index.reshape(2, NW, CHUNKS, K)
    zeros1 = jnp.zeros((R_T, _DW), jnp.float32)
    ones_k = jnp.ones((K, _DW), jnp.float32)
    zeros128 = jnp.zeros((R_T, 128), jnp.float32)
    zeros64 = jnp.zeros((R_T, 64), jnp.float32)

    deg = _sc_degree()(ei, ones_k, zeros1)
    h1 = _tc0(x, W1)
    g1, dinv = _tc1(h1, deg)
    acc1 = _sc_scatter(128, K, NB1)(g1, ei, zeros128)
    g2 = _tc2(acc1, g1, dinv, b1.reshape(1, 128), W2)
    ei80 = edge_index.reshape(2, NW, E_W // K2, K2)
    acc2 = _sc_scatter(64, K2, NB2)(g2, ei80, zeros64)
    out = _tc3(acc2, g2, dinv, b2.reshape(1, 64), Wfc,
               bfc.reshape(1, 1))
    return out.reshape(-1)


# speedup vs baseline: 1.1593x; 1.0915x over previous
import functools

import jax
import jax.numpy as jnp
from jax import lax
from jax.experimental import pallas as pl
from jax.experimental.pallas import tpu as pltpu
from jax.experimental.pallas import tpu_sc as plsc

N = 10000
E = 320000
NC, NS = 2, 16
NW = NC * NS
E_W = E // NW
K = 40
CHUNKS = E_W // K
N_PAD = 10240
R_T = N_PAD // NS


def _sc_scatter(D, Kc, nbuf):
    ch = E_W // Kc
    assert ch * Kc == E_W and ch % nbuf == 0
    mesh = plsc.VectorSubcoreMesh(core_axis_name="c", subcore_axis_name="s")

    @functools.partial(
        pl.kernel,
        out_type=jax.ShapeDtypeStruct((NC, N_PAD, D), jnp.float32),
        mesh=mesh,
        compiler_params=pltpu.CompilerParams(use_tc_tiling_on_sc=False),
        scratch_types=[
            pltpu.VMEM((ch, Kc), jnp.int32),
            pltpu.VMEM((ch, Kc), jnp.int32),
            [pltpu.VMEM((Kc, D), jnp.float32) for _ in range(nbuf)],
            pltpu.VMEM_SHARED((N_PAD, D), jnp.float32),
            [pltpu.SemaphoreType.DMA for _ in range(nbuf)],
        ],
    )
    def k(g_hbm, ei_hbm, zeros_hbm, out_hbm, src_v, dst_v, rows_v,
          acc_s, sems):
        c = lax.axis_index("c")
        s = lax.axis_index("s")
        w = s * NC + c
        row0 = pl.multiple_of(s * R_T, 8)
        pltpu.sync_copy(zeros_hbm, acc_s.at[pl.ds(row0, R_T)])
        pltpu.sync_copy(ei_hbm.at[0, w], src_v)
        pltpu.sync_copy(ei_hbm.at[1, w], dst_v)
        plsc.subcore_barrier()

        for b in range(nbuf - 1):
            pltpu.async_copy(g_hbm.at[src_v.at[b]], rows_v[b], sems[b])

        def body(jo, carry):
            for b in range(nbuf):
                j = jo * nbuf + b
                pltpu.make_async_copy(g_hbm.at[src_v.at[j]], rows_v[b],
                                      sems[b]).wait()
                pltpu.sync_copy(rows_v[b], acc_s.at[dst_v.at[j]], add=True)
                jn = j + nbuf - 1
                bn = (b + nbuf - 1) % nbuf

                @pl.when(jn < ch)
                def _():
                    pltpu.async_copy(g_hbm.at[src_v.at[jn]], rows_v[bn],
                                     sems[bn])
            return carry

        lax.fori_loop(0, ch // nbuf, body, 0)
        plsc.subcore_barrier()
        pltpu.sync_copy(acc_s.at[pl.ds(row0, R_T)],
                        out_hbm.at[c, pl.ds(row0, R_T)])

    return k


_DW = 16
_KD = 80
_CHD = E_W // _KD


def _sc_degree():
    mesh = plsc.VectorSubcoreMesh(core_axis_name="c", subcore_axis_name="s")

    @functools.partial(
        pl.kernel,
        out_type=jax.ShapeDtypeStruct((NC, N_PAD, _DW), jnp.float32),
        mesh=mesh,
        compiler_params=pltpu.CompilerParams(use_tc_tiling_on_sc=False),
        scratch_types=[
            pltpu.VMEM((_CHD, _KD), jnp.int32),
            pltpu.VMEM((_KD, _DW), jnp.float32),
            pltpu.VMEM_SHARED((N_PAD, _DW), jnp.float32),
            pltpu.SemaphoreType.DMA,
        ],
    )
    def k(ei_hbm, ones_hbm, zeros_hbm, out_hbm, dst_v, ones_v, deg_s, sem):
        c = lax.axis_index("c")
        s = lax.axis_index("s")
        w = s * NC + c
        row0 = pl.multiple_of(s * R_T, 8)
        pltpu.sync_copy(zeros_hbm, deg_s.at[pl.ds(row0, R_T)])
        pltpu.sync_copy(ones_hbm, ones_v)
        pltpu.sync_copy(ei_hbm.at[1, w], dst_v)
        plsc.subcore_barrier()

        def fire(j, carry):
            pltpu.async_copy(ones_v, deg_s.at[dst_v.at[j]], sem, add=True)
            return carry

        lax.fori_loop(0, _CHD, fire, 0)

        def drain(j, carry):
            pltpu.make_async_copy(ones_v, deg_s.at[dst_v.at[0]], sem).wait()
            return carry

        lax.fori_loop(0, _CHD, drain, 0)
        plsc.subcore_barrier()
        pltpu.sync_copy(deg_s.at[pl.ds(row0, R_T)],
                        out_hbm.at[c, pl.ds(row0, R_T)])

    return k


_BR = 1000


def _tc0(x, W1):
    def body(x_ref, w_ref, h_ref):
        h_ref[...] = jnp.dot(x_ref[...], w_ref[...],
                             preferred_element_type=jnp.float32)

    return pl.pallas_call(
        body,
        grid=(N // _BR,),
        in_specs=[
            pl.BlockSpec((_BR, 128), lambda i: (i, 0)),
            pl.BlockSpec((128, 128), lambda i: (0, 0)),
        ],
        out_specs=pl.BlockSpec((_BR, 128), lambda i: (i, 0)),
        out_shape=jax.ShapeDtypeStruct((N, 128), jnp.float32),
    )(x, W1)


def _tc1(h1, deg):
    def body(h_ref, d0_ref, d1_ref, g_ref, dinv_ref):
        deg_tot = d0_ref[0][:, 0:1] + d1_ref[0][:, 0:1] + 1.0
        dinv = lax.rsqrt(deg_tot)
        g_ref[...] = h_ref[...] * dinv
        dinv_ref[...] = dinv

    return pl.pallas_call(
        body,
        grid=(N // _BR,),
        in_specs=[
            pl.BlockSpec((_BR, 128), lambda i: (i, 0)),
            pl.BlockSpec((1, _BR, _DW), lambda i: (0, i, 0)),
            pl.BlockSpec((1, _BR, _DW), lambda i: (1, i, 0)),
        ],
        out_specs=[
            pl.BlockSpec((_BR, 128), lambda i: (i, 0)),
            pl.BlockSpec((_BR, 1), lambda i: (i, 0)),
        ],
        out_shape=[
            jax.ShapeDtypeStruct((N, 128), jnp.float32),
            jax.ShapeDtypeStruct((N, 1), jnp.float32),
        ],
    )(h1, deg, deg)


def _tc2(acc1, g1, dinv, b1, W2):
    def body(a0_ref, a1_ref, g_ref, dinv_ref, b_ref, w_ref, g2_ref):
        z = dinv_ref[...] * (a0_ref[0] + a1_ref[0] + g_ref[...]) + b_ref[...]
        z = jnp.maximum(z, 0.0)
        g2_ref[...] = (
            jnp.dot(z, w_ref[...], preferred_element_type=jnp.float32)
            * dinv_ref[...])

    return pl.pallas_call(
        body,
        grid=(N // _BR,),
        in_specs=[
            pl.BlockSpec((1, _BR, 128), lambda i: (0, i, 0)),
            pl.BlockSpec((1, _BR, 128), lambda i: (1, i, 0)),
            pl.BlockSpec((_BR, 128), lambda i: (i, 0)),
            pl.BlockSpec((_BR, 1), lambda i: (i, 0)),
            pl.BlockSpec((1, 128), lambda i: (0, 0)),
            pl.BlockSpec((128, 64), lambda i: (0, 0)),
        ],
        out_specs=pl.BlockSpec((_BR, 64), lambda i: (i, 0)),
        out_shape=jax.ShapeDtypeStruct((N, 64), jnp.float32),
    )(acc1, acc1, g1, dinv, b1, W2)


def _tc3(acc2, g2, dinv, b2, Wfc, bfc):
    def body(a0_ref, a1_ref, g_ref, dinv_ref, b_ref, w_ref, bfc_ref, o_ref):
        z = dinv_ref[...] * (a0_ref[0] + a1_ref[0] + g_ref[...]) + b_ref[...]
        z = jnp.maximum(z, 0.0)
        o_ref[...] = (
            jnp.dot(z, w_ref[...], preferred_element_type=jnp.float32)
            + bfc_ref[...])[:, 0]

    return pl.pallas_call(
        body,
        grid=(1,),
        in_specs=[
            pl.BlockSpec((1, N, 64), lambda i: (0, 0, 0)),
            pl.BlockSpec((1, N, 64), lambda i: (1, 0, 0)),
            pl.BlockSpec((N, 64), lambda i: (0, 0)),
            pl.BlockSpec((N, 1), lambda i: (0, 0)),
            pl.BlockSpec((1, 64), lambda i: (0, 0)),
            pl.BlockSpec((64, 1), lambda i: (0, 0)),
            pl.BlockSpec((1, 1), lambda i: (0, 0)),
        ],
        out_specs=pl.BlockSpec((N,), lambda i: (0,)),
        out_shape=jax.ShapeDtypeStruct((N,), jnp.float32),
    )(acc2, acc2, g2, dinv, b2, Wfc, bfc)


NB1 = 5
K2, NB2 = 80, 5


def kernel(x, edge_index, W1, b1, W2, b2, Wfc, bfc):
    ei = edge_index.reshape(2, NW, CHUNKS, K)
    ei80 = edge_index.reshape(2, NW, E_W // K2, K2)
    zeros1 = jnp.zeros((R_T, _DW), jnp.float32)
    ones_k = jnp.ones((_KD, _DW), jnp.float32)
    zeros128 = jnp.zeros((R_T, 128), jnp.float32)
    zeros64 = jnp.zeros((R_T, 64), jnp.float32)

    deg = _sc_degree()(ei80, ones_k, zeros1)
    h1 = _tc0(x, W1)
    g1, dinv = _tc1(h1, deg)
    acc1 = _sc_scatter(128, K, NB1)(g1, ei, zeros128)
    g2 = _tc2(acc1, g1, dinv, b1.reshape(1, 128), W2)
    acc2 = _sc_scatter(64, K2, NB2)(g2, ei80, zeros64)
    return _tc3(acc2, g2, dinv, b2.reshape(1, 64), Wfc,
                bfc.reshape(1, 1))


# speedup vs baseline: 1.1595x; 1.0001x over previous
import functools

import jax
import jax.numpy as jnp
from jax import lax
from jax.experimental import pallas as pl
from jax.experimental.pallas import tpu as pltpu
from jax.experimental.pallas import tpu_sc as plsc

N = 10000
E = 320000
NC, NS = 2, 16
NW = NC * NS
E_W = E // NW
K = 40
CHUNKS = E_W // K
N_PAD = 10240
R_T = N_PAD // NS


def _sc_scatter(D, Kc, nbuf):
    ch = E_W // Kc
    assert ch * Kc == E_W and ch % nbuf == 0
    mesh = plsc.VectorSubcoreMesh(core_axis_name="c", subcore_axis_name="s")

    @functools.partial(
        pl.kernel,
        out_type=jax.ShapeDtypeStruct((NC, N_PAD, D), jnp.float32),
        mesh=mesh,
        compiler_params=pltpu.CompilerParams(use_tc_tiling_on_sc=False),
        scratch_types=[
            pltpu.VMEM((ch, Kc), jnp.int32),
            pltpu.VMEM((ch, Kc), jnp.int32),
            [pltpu.VMEM((Kc, D), jnp.float32) for _ in range(nbuf)],
            pltpu.VMEM_SHARED((N_PAD, D), jnp.float32),
            [pltpu.SemaphoreType.DMA for _ in range(nbuf)],
        ],
    )
    def k(g_hbm, ei_hbm, zeros_hbm, out_hbm, src_v, dst_v, rows_v,
          acc_s, sems):
        c = lax.axis_index("c")
        s = lax.axis_index("s")
        w = s * NC + c
        row0 = pl.multiple_of(s * R_T, 8)
        pltpu.sync_copy(zeros_hbm, acc_s.at[pl.ds(row0, R_T)])
        pltpu.sync_copy(ei_hbm.at[0, w], src_v)
        pltpu.sync_copy(ei_hbm.at[1, w], dst_v)
        plsc.subcore_barrier()

        for b in range(nbuf - 1):
            pltpu.async_copy(g_hbm.at[src_v.at[b]], rows_v[b], sems[b])

        def body(jo, carry):
            for b in range(nbuf):
                j = jo * nbuf + b
                pltpu.make_async_copy(g_hbm.at[src_v.at[j]], rows_v[b],
                                      sems[b]).wait()
                pltpu.sync_copy(rows_v[b], acc_s.at[dst_v.at[j]], add=True)
                jn = j + nbuf - 1
                bn = (b + nbuf - 1) % nbuf

                @pl.when(jn < ch)
                def _():
                    pltpu.async_copy(g_hbm.at[src_v.at[jn]], rows_v[bn],
                                     sems[bn])
            return carry

        lax.fori_loop(0, ch // nbuf, body, 0)
        plsc.subcore_barrier()
        pltpu.sync_copy(acc_s.at[pl.ds(row0, R_T)],
                        out_hbm.at[c, pl.ds(row0, R_T)])

    return k


_DW = 16
_KD = 80
_CHD = E_W // _KD


def _sc_degree():
    mesh = plsc.VectorSubcoreMesh(core_axis_name="c", subcore_axis_name="s")

    @functools.partial(
        pl.kernel,
        out_type=jax.ShapeDtypeStruct((NC, N_PAD, _DW), jnp.float32),
        mesh=mesh,
        compiler_params=pltpu.CompilerParams(use_tc_tiling_on_sc=False),
        scratch_types=[
            pltpu.VMEM((_CHD, _KD), jnp.int32),
            pltpu.VMEM((_KD, _DW), jnp.float32),
            pltpu.VMEM_SHARED((N_PAD, _DW), jnp.float32),
            pltpu.SemaphoreType.DMA,
        ],
    )
    def k(ei_hbm, ones_hbm, zeros_hbm, out_hbm, dst_v, ones_v, deg_s, sem):
        c = lax.axis_index("c")
        s = lax.axis_index("s")
        w = s * NC + c
        row0 = pl.multiple_of(s * R_T, 8)
        pltpu.sync_copy(zeros_hbm, deg_s.at[pl.ds(row0, R_T)])
        pltpu.sync_copy(ones_hbm, ones_v)
        pltpu.sync_copy(ei_hbm.at[1, w], dst_v)
        plsc.subcore_barrier()

        def fire(j, carry):
            pltpu.async_copy(ones_v, deg_s.at[dst_v.at[j]], sem, add=True)
            return carry

        lax.fori_loop(0, _CHD, fire, 0)

        def drain(j, carry):
            pltpu.make_async_copy(ones_v, deg_s.at[dst_v.at[0]], sem).wait()
            return carry

        lax.fori_loop(0, _CHD, drain, 0)
        plsc.subcore_barrier()
        pltpu.sync_copy(deg_s.at[pl.ds(row0, R_T)],
                        out_hbm.at[c, pl.ds(row0, R_T)])

    return k


_BR = 1000
_BR2 = 2000


def _tc0(x, W1):
    def body(x_ref, w_ref, h_ref):
        h_ref[...] = jnp.dot(x_ref[...], w_ref[...],
                             preferred_element_type=jnp.float32)

    return pl.pallas_call(
        body,
        grid=(N // _BR,),
        in_specs=[
            pl.BlockSpec((_BR, 128), lambda i: (i, 0)),
            pl.BlockSpec((128, 128), lambda i: (0, 0)),
        ],
        out_specs=pl.BlockSpec((_BR, 128), lambda i: (i, 0)),
        out_shape=jax.ShapeDtypeStruct((N, 128), jnp.float32),
    )(x, W1)


_DR = N_PAD * _DW // 128


def _tc1(h1, deg):
    def body(h_ref, d0_ref, d1_ref, g_ref, dinv_ref):
        deg_tot = d0_ref[0][:, 0:1] + d1_ref[0][:, 0:1] + 1.0
        dinv = lax.rsqrt(deg_tot)
        g_ref[...] = h_ref[...] * dinv
        dinv_ref[...] = dinv

    return pl.pallas_call(
        body,
        grid=(N // _BR,),
        in_specs=[
            pl.BlockSpec((_BR, 128), lambda i: (i, 0)),
            pl.BlockSpec((1, _BR, _DW), lambda i: (0, i, 0)),
            pl.BlockSpec((1, _BR, _DW), lambda i: (1, i, 0)),
        ],
        out_specs=[
            pl.BlockSpec((_BR, 128), lambda i: (i, 0)),
            pl.BlockSpec((_BR, 1), lambda i: (i, 0)),
        ],
        out_shape=[
            jax.ShapeDtypeStruct((N, 128), jnp.float32),
            jax.ShapeDtypeStruct((N, 1), jnp.float32),
        ],
    )(h1, deg, deg)


def _tc2(acc1, g1, dinv, b1, W2):
    def body(a0_ref, a1_ref, g_ref, dinv_ref, b_ref, w_ref, g2_ref):
        z = dinv_ref[...] * (a0_ref[0] + a1_ref[0] + g_ref[...]) + b_ref[...]
        z = jnp.maximum(z, 0.0)
        g2_ref[...] = (
            jnp.dot(z, w_ref[...], preferred_element_type=jnp.float32)
            * dinv_ref[...])

    return pl.pallas_call(
        body,
        grid=(N // _BR,),
        in_specs=[
            pl.BlockSpec((1, _BR, 128), lambda i: (0, i, 0)),
            pl.BlockSpec((1, _BR, 128), lambda i: (1, i, 0)),
            pl.BlockSpec((_BR, 128), lambda i: (i, 0)),
            pl.BlockSpec((_BR, 1), lambda i: (i, 0)),
            pl.BlockSpec((1, 128), lambda i: (0, 0)),
            pl.BlockSpec((128, 64), lambda i: (0, 0)),
        ],
        out_specs=pl.BlockSpec((_BR, 64), lambda i: (i, 0)),
        out_shape=jax.ShapeDtypeStruct((N, 64), jnp.float32),
    )(acc1, acc1, g1, dinv, b1, W2)


def _tc3(acc2, g2, dinv, b2, Wfc, bfc):
    def body(a0_ref, a1_ref, g_ref, dinv_ref, b_ref, w_ref, bfc_ref, o_ref):
        z = dinv_ref[...] * (a0_ref[0] + a1_ref[0] + g_ref[...]) + b_ref[...]
        z = jnp.maximum(z, 0.0)
        o_ref[...] = (
            jnp.dot(z, w_ref[...], preferred_element_type=jnp.float32)
            + bfc_ref[...])[:, 0]

    return pl.pallas_call(
        body,
        grid=(1,),
        in_specs=[
            pl.BlockSpec((1, N, 64), lambda i: (0, 0, 0)),
            pl.BlockSpec((1, N, 64), lambda i: (1, 0, 0)),
            pl.BlockSpec((N, 64), lambda i: (0, 0)),
            pl.BlockSpec((N, 1), lambda i: (0, 0)),
            pl.BlockSpec((1, 64), lambda i: (0, 0)),
            pl.BlockSpec((64, 1), lambda i: (0, 0)),
            pl.BlockSpec((1, 1), lambda i: (0, 0)),
        ],
        out_specs=pl.BlockSpec((N,), lambda i: (0,)),
        out_shape=jax.ShapeDtypeStruct((N,), jnp.float32),
    )(acc2, acc2, g2, dinv, b2, Wfc, bfc)


NB1 = 5
K2, NB2 = 80, 5


def kernel(x, edge_index, W1, b1, W2, b2, Wfc, bfc):
    ei = edge_index.reshape(2, NW, CHUNKS, K)
    ei80 = edge_index.reshape(2, NW, E_W // K2, K2)
    zeros1 = jnp.zeros((R_T, _DW), jnp.float32)
    ones_k = jnp.ones((_KD, _DW), jnp.float32)
    zeros128 = jnp.zeros((R_T, 128), jnp.float32)
    zeros64 = jnp.zeros((R_T, 64), jnp.float32)

    deg = _sc_degree()(ei80, ones_k, zeros1)
    h1 = _tc0(x, W1)
    g1, dinv = _tc1(h1, deg)
    acc1 = _sc_scatter(128, K, NB1)(g1, ei, zeros128)
    g2 = _tc2(acc1, g1, dinv, b1.reshape(1, 128), W2)
    acc2 = _sc_scatter(64, K2, NB2)(g2, ei80, zeros64)
    return _tc3(acc2, g2, dinv, b2.reshape(1, 64), Wfc,
                bfc.reshape(1, 1))
